# baseline (speedup 1.0000x reference)
import functools

import jax
import jax.numpy as jnp
from jax import lax
from jax.experimental import pallas as pl
from jax.experimental.pallas import tpu as pltpu
from jax.experimental.pallas import tpu_sc as plsc

N_NODES = 10000
N_EDGES = 320000
NPAD = 10240

NC = 2
NS = 16
NW = NC * NS
BATCH = 80
ROWS_PER_TILE = NPAD // NS

EPT_W = N_EDGES // NW
NB_W = EPT_W // BATCH
EPT_S = N_EDGES // NS
BATCH_S = 40
NB_S = EPT_S // BATCH_S

_mesh = plsc.VectorSubcoreMesh(core_axis_name="c", subcore_axis_name="s")
_sc_params = pltpu.CompilerParams(use_tc_tiling_on_sc=False)


_DEG_NBUF = 5
_DEG_ROUNDS = NB_W // _DEG_NBUF


@functools.partial(
    pl.kernel,
    out_type=(
        jax.ShapeDtypeStruct((NC, NPAD, 8), jnp.float32),
        jax.ShapeDtypeStruct((NC, NPAD, 8), jnp.float32),
    ),
    mesh=_mesh,
    scratch_types=[
        pltpu.VMEM((NB_W, BATCH), jnp.int32),
        pltpu.VMEM((NB_W, BATCH), jnp.int32),
        pltpu.VMEM((BATCH, 8), jnp.float32),
        pltpu.VMEM_SHARED((NPAD, 8), jnp.float32),
        pltpu.VMEM_SHARED((NPAD, 8), jnp.float32),
    ] + [pltpu.SemaphoreType.DMA] * (2 * _DEG_NBUF),
    compiler_params=_sc_params,
)
def _sc_degrees(src_h, dst_h, ones_h, z8_h, out_o, out_i,
                idx_s, idx_d, ones_v, acc_o, acc_i, *sems):
    osem = sems[:_DEG_NBUF]
    isem = sems[_DEG_NBUF:]
    c = lax.axis_index("c")
    s = lax.axis_index("s")
    wid = c * NS + s
    r0 = s * ROWS_PER_TILE
    pltpu.sync_copy(z8_h, acc_o.at[pl.ds(r0, ROWS_PER_TILE)])
    pltpu.sync_copy(z8_h, acc_i.at[pl.ds(r0, ROWS_PER_TILE)])
    pltpu.sync_copy(src_h.at[wid], idx_s)
    pltpu.sync_copy(dst_h.at[wid], idx_d)
    pltpu.sync_copy(ones_h, ones_v)
    plsc.subcore_barrier()

    def round_body(g, carry):
        handles = []
        for b in range(_DEG_NBUF):
            i = g * _DEG_NBUF + b
            handles.append(pltpu.async_copy(
                ones_v, acc_o.at[idx_s.at[i]], osem[b], add=True))
            handles.append(pltpu.async_copy(
                ones_v, acc_i.at[idx_d.at[i]], isem[b], add=True))
        for h in handles:
            h.wait()
        return carry

    lax.fori_loop(0, _DEG_ROUNDS, round_body, 0)
    plsc.subcore_barrier()
    pltpu.sync_copy(acc_o.at[pl.ds(r0, ROWS_PER_TILE)],
                    out_o.at[c, pl.ds(r0, ROWS_PER_TILE)])
    pltpu.sync_copy(acc_i.at[pl.ds(r0, ROWS_PER_TILE)],
                    out_i.at[c, pl.ds(r0, ROWS_PER_TILE)])


_B_NBUF = 10
_B_ROUNDS = NB_S // _B_NBUF


@functools.partial(
    pl.kernel,
    out_type=jax.ShapeDtypeStruct((NC, NPAD, 64), jnp.float32),
    mesh=_mesh,
    scratch_types=[
        pltpu.VMEM((_B_NBUF, 2, BATCH_S), jnp.int32),
        pltpu.VMEM((_B_NBUF, BATCH_S, 64), jnp.float32),
        pltpu.VMEM_SHARED((NPAD, 64), jnp.float32),
    ] + [pltpu.SemaphoreType.DMA] * (3 * _B_NBUF),
    compiler_params=_sc_params,
)
def _sc_agg_colsplit(e2_h, tbl_h, z_h, out_h, ring, rows, acc, *sems):
    gsem = sems[:_B_NBUF]
    ssem = sems[_B_NBUF:2 * _B_NBUF]
    isem = sems[2 * _B_NBUF:]
    c = lax.axis_index("c")
    s = lax.axis_index("s")
    r0 = s * ROWS_PER_TILE
    for r in range(ROWS_PER_TILE // 128):
        pltpu.sync_copy(z_h, acc.at[pl.ds(r0 + r * 128, 128)])
    plsc.subcore_barrier()

    for b in range(_B_NBUF):
        pltpu.async_copy(e2_h.at[c, s, b], ring.at[b], isem[b])
    for b in range(_B_NBUF):
        pltpu.make_async_copy(e2_h.at[c, s, b], ring.at[b], isem[b]).wait()
        pltpu.async_copy(tbl_h.at[ring.at[b, 0]], rows.at[b], gsem[b])

    def round_body(g, carry):
        handles = []
        for b in range(_B_NBUF):
            i = g * _B_NBUF + b
            nxt = i + _B_NBUF
            pltpu.make_async_copy(
                tbl_h.at[ring.at[b, 0]], rows.at[b], gsem[b]).wait()
            handles.append(pltpu.async_copy(
                rows.at[b], acc.at[ring.at[b, 1]], ssem[b], add=True))

            @pl.when(nxt < NB_S)
            def _():
                pltpu.async_copy(e2_h.at[c, s, nxt], ring.at[b], isem[b])
        for b in range(_B_NBUF):
            i = g * _B_NBUF + b
            nxt = i + _B_NBUF
            handles[b].wait()

            @pl.when(nxt < NB_S)
            def _():
                pltpu.make_async_copy(
                    e2_h.at[c, s, nxt], ring.at[b], isem[b]).wait()
                pltpu.async_copy(
                    tbl_h.at[ring.at[b, 0]], rows.at[b], gsem[b])
        return carry

    lax.fori_loop(0, _B_ROUNDS, round_body, 0)
    plsc.subcore_barrier()
    pltpu.sync_copy(acc.at[pl.ds(r0, ROWS_PER_TILE)],
                    out_h.at[c, pl.ds(r0, ROWS_PER_TILE)])


_C_NBUF = 5
_C_ROUNDS = NB_W // _C_NBUF


@functools.partial(
    pl.kernel,
    out_type=jax.ShapeDtypeStruct((NC, NPAD, 32), jnp.float32),
    mesh=_mesh,
    scratch_types=[
        pltpu.VMEM((NB_W, BATCH), jnp.int32),
        pltpu.VMEM((NB_W, BATCH), jnp.int32),
        pltpu.VMEM((_C_NBUF, BATCH, 32), jnp.float32),
        pltpu.VMEM_SHARED((NPAD, 32), jnp.float32),
    ] + [pltpu.SemaphoreType.DMA] * (2 * _C_NBUF),
    compiler_params=_sc_params,
)
def _sc_agg32(src_h, dst_h, tbl_h, z_h, out_h,
              idx_g, idx_c, rows, acc, *sems):
    gsem = sems[:_C_NBUF]
    ssem = sems[_C_NBUF:]
    c = lax.axis_index("c")
    s = lax.axis_index("s")
    wid = c * NS + s
    r0 = s * ROWS_PER_TILE
    for r in range(ROWS_PER_TILE // 128):
        pltpu.sync_copy(z_h, acc.at[pl.ds(r0 + r * 128, 128)])
    pltpu.sync_copy(src_h.at[wid], idx_g)
    pltpu.sync_copy(dst_h.at[wid], idx_c)
    plsc.subcore_barrier()

    for b in range(_C_NBUF):
        pltpu.async_copy(tbl_h.at[idx_g.at[b]], rows.at[b], gsem[b])

    def round_body(g, carry):
        handles = []
        for b in range(_C_NBUF):
            i = g * _C_NBUF + b
            pltpu.make_async_copy(
                tbl_h.at[idx_g.at[i]], rows.at[b], gsem[b]).wait()
            handles.append(pltpu.async_copy(
                rows.at[b], acc.at[idx_c.at[i]], ssem[b], add=True))
        for b in range(_C_NBUF):
            handles[b].wait()
            nxt = (g + 1) * _C_NBUF + b

            @pl.when(nxt < NB_W)
            def _():
                pltpu.async_copy(
                    tbl_h.at[idx_g.at[nxt]], rows.at[b], gsem[b])
        return carry

    lax.fori_loop(0, _C_ROUNDS, round_body, 0)
    plsc.subcore_barrier()
    pltpu.sync_copy(acc.at[pl.ds(r0, ROWS_PER_TILE)],
                    out_h.at[c, pl.ds(r0, ROWS_PER_TILE)])


def _tc_norms_body(dego_ref, degi_ref, x_ref, xs_ref, ns_ref, nd_ref):
    do = (dego_ref[0] + dego_ref[1])[:, :1]
    di = (degi_ref[0] + degi_ref[1])[:, :1]
    ns = lax.rsqrt(jnp.maximum(do, 1.0))
    nd = lax.rsqrt(jnp.maximum(di, 1.0))
    ns_ref[...] = ns
    nd_ref[...] = nd
    xs = x_ref[...] * ns
    xs_ref[0] = xs[:, :64]
    xs_ref[1] = xs[:, 64:]


def _tc_norms(deg_o, deg_i, x_pad):
    blk = 2048
    grid = NPAD // blk
    return pl.pallas_call(
        _tc_norms_body,
        grid=(grid,),
        in_specs=[
            pl.BlockSpec((NC, blk, 8), lambda i: (0, i, 0)),
            pl.BlockSpec((NC, blk, 8), lambda i: (0, i, 0)),
            pl.BlockSpec((blk, 128), lambda i: (i, 0)),
        ],
        out_specs=[
            pl.BlockSpec((2, blk, 64), lambda i: (0, i, 0)),
            pl.BlockSpec((blk, 1), lambda i: (i, 0)),
            pl.BlockSpec((blk, 1), lambda i: (i, 0)),
        ],
        out_shape=[
            jax.ShapeDtypeStruct((2, NPAD, 64), jnp.float32),
            jax.ShapeDtypeStruct((NPAD, 1), jnp.float32),
            jax.ShapeDtypeStruct((NPAD, 1), jnp.float32),
        ],
    )(deg_o, deg_i, x_pad)


def _tc_mid_body(a_ref, nd_ref, ns_ref, w1_ref, b1_ref, w2_ref, t_ref):
    agg = jnp.concatenate([a_ref[0], a_ref[1]], axis=1) * nd_ref[...]
    h = jnp.dot(agg, w1_ref[...], preferred_element_type=jnp.float32)
    h = jnp.maximum(h + b1_ref[...][None, :], 0.0)
    t_ref[...] = jnp.dot(h * ns_ref[...], w2_ref[...],
                         preferred_element_type=jnp.float32)


def _tc_mid(agg0, nd, ns, W1, b1, W2):
    blk = 1024
    grid = NPAD // blk
    return pl.pallas_call(
        _tc_mid_body,
        grid=(grid,),
        in_specs=[
            pl.BlockSpec((NC, blk, 64), lambda i: (0, i, 0)),
            pl.BlockSpec((blk, 1), lambda i: (i, 0)),
            pl.BlockSpec((blk, 1), lambda i: (i, 0)),
            pl.BlockSpec((128, 256), lambda i: (0, 0)),
            pl.BlockSpec((256,), lambda i: (0,)),
            pl.BlockSpec((256, 32), lambda i: (0, 0)),
        ],
        out_specs=pl.BlockSpec((blk, 32), lambda i: (i, 0)),
        out_shape=jax.ShapeDtypeStruct((NPAD, 32), jnp.float32),
    )(agg0, nd, ns, W1, b1, W2)


def _tc_final_body(a_ref, nd_ref, b2_ref, o_ref):
    agg = (a_ref[0] + a_ref[1]) * nd_ref[...]
    o_ref[...] = jax.nn.sigmoid(agg + b2_ref[...][None, :])


def _tc_final(agg1, nd, b2):
    blk = 1024
    grid = NPAD // blk
    return pl.pallas_call(
        _tc_final_body,
        grid=(grid,),
        in_specs=[
            pl.BlockSpec((NC, blk, 32), lambda i: (0, i, 0)),
            pl.BlockSpec((blk, 1), lambda i: (i, 0)),
            pl.BlockSpec((32,), lambda i: (0,)),
        ],
        out_specs=pl.BlockSpec((blk, 32), lambda i: (i, 0)),
        out_shape=jax.ShapeDtypeStruct((NPAD, 32), jnp.float32),
    )(agg1, nd, b2)


def kernel(x, edge_index, W1, b1, W2, b2):
    src = edge_index[0].astype(jnp.int32)
    dst = edge_index[1].astype(jnp.int32)
    src_w = src.reshape(NW, NB_W, BATCH)
    dst_w = dst.reshape(NW, NB_W, BATCH)
    src_s = src.reshape(NS, NB_S, BATCH_S)
    dst_s = dst.reshape(NS, NB_S, BATCH_S)
    e2 = jnp.stack([jnp.stack([src_s, dst_s], axis=2),
                    jnp.stack([src_s + NPAD, dst_s], axis=2)])

    ones8 = jnp.ones((BATCH, 8), jnp.float32)
    z8 = jnp.zeros((ROWS_PER_TILE, 8), jnp.float32)
    deg_o, deg_i = _sc_degrees(src_w, dst_w, ones8, z8)

    x_pad = jnp.pad(x, ((0, NPAD - N_NODES), (0, 0)))
    xs_pair, ns, nd = _tc_norms(deg_o, deg_i, x_pad)

    z64 = jnp.zeros((128, 64), jnp.float32)
    agg0 = _sc_agg_colsplit(e2, xs_pair.reshape(2 * NPAD, 64), z64)

    t = _tc_mid(agg0, nd, ns, W1, b1, W2)

    z32 = jnp.zeros((128, 32), jnp.float32)
    agg1 = _sc_agg32(src_w, dst_w, t, z32)

    out = _tc_final(agg1, nd, b2)
    return out[:N_NODES]


# speedup vs baseline: 1.0096x; 1.0096x over previous
import functools

import jax
import jax.numpy as jnp
from jax import lax
from jax.experimental import pallas as pl
from jax.experimental.pallas import tpu as pltpu
from jax.experimental.pallas import tpu_sc as plsc

N_NODES = 10000
N_EDGES = 320000
NPAD = 10240

NC = 2
NS = 16
NW = NC * NS
BATCH = 80
ROWS_PER_TILE = NPAD // NS

EPT_W = N_EDGES // NW
NB_W = EPT_W // BATCH
EPT_S = N_EDGES // NS
BATCH_S = 80
NB_S = EPT_S // BATCH_S

_mesh = plsc.VectorSubcoreMesh(core_axis_name="c", subcore_axis_name="s")
_sc_params = pltpu.CompilerParams(use_tc_tiling_on_sc=False)
_sc_prep_params = pltpu.CompilerParams(
    use_tc_tiling_on_sc=False, needs_layout_passes=False)


_DEG_NBUF = 5
_A_ROUNDS = NB_S // _DEG_NBUF
_MAGIC = jnp.int32(0x5F3759DF)


def _rsqrt16(v):
    v = jnp.maximum(v, 1.0)
    y = plsc.bitcast(_MAGIC - lax.shift_right_logical(
        plsc.bitcast(v, jnp.int32), 1), jnp.float32)
    for _ in range(3):
        y = y * (1.5 - 0.5 * v * y * y)
    return y


@functools.partial(
    pl.kernel,
    out_type=(
        jax.ShapeDtypeStruct((2 * NPAD, 64), jnp.float32),
        jax.ShapeDtypeStruct((NPAD, 16), jnp.float32),
        jax.ShapeDtypeStruct((NPAD, 16), jnp.float32),
    ),
    mesh=_mesh,
    scratch_types=[
        pltpu.VMEM((_DEG_NBUF, 2, BATCH_S), jnp.int32),
        pltpu.VMEM((BATCH_S, 16), jnp.float32),
        pltpu.VMEM((ROWS_PER_TILE, 16), jnp.float32),
        pltpu.VMEM((ROWS_PER_TILE, 16), jnp.float32),
        pltpu.VMEM((80, 128), jnp.float32),
        pltpu.VMEM((80, 64), jnp.float32),
        pltpu.VMEM_SHARED((NPAD, 16), jnp.float32),
        pltpu.VMEM_SHARED((NPAD, 16), jnp.float32),
    ] + [pltpu.SemaphoreType.DMA] * (3 * _DEG_NBUF),
    compiler_params=_sc_prep_params,
)
def _sc_prep(e2_h, x_h, ones_h, z16_h, xs_h, ns_h, nd_h,
             ring, ones_v, nsbuf, ndbuf, xbuf, obuf, acc_o, acc_i, *sems):
    osem = sems[:_DEG_NBUF]
    dsem = sems[_DEG_NBUF:2 * _DEG_NBUF]
    isem = sems[2 * _DEG_NBUF:]
    c = lax.axis_index("c")
    s = lax.axis_index("s")
    r0 = s * ROWS_PER_TILE
    pltpu.sync_copy(z16_h, acc_o.at[pl.ds(r0, ROWS_PER_TILE)])
    pltpu.sync_copy(z16_h, acc_i.at[pl.ds(r0, ROWS_PER_TILE)])
    pltpu.sync_copy(ones_h, ones_v)
    plsc.subcore_barrier()

    for b in range(_DEG_NBUF):
        pltpu.async_copy(e2_h.at[0, s, b], ring.at[b], isem[b])

    def round_body(g, carry):
        handles = []
        for b in range(_DEG_NBUF):
            i = g * _DEG_NBUF + b
            pltpu.make_async_copy(
                e2_h.at[0, s, i], ring.at[b], isem[b]).wait()
            handles.append(pltpu.async_copy(
                ones_v, acc_o.at[ring.at[b, 0]], osem[b], add=True))
            handles.append(pltpu.async_copy(
                ones_v, acc_i.at[ring.at[b, 1]], dsem[b], add=True))
        for b in range(_DEG_NBUF):
            i = g * _DEG_NBUF + b
            nxt = i + _DEG_NBUF
            handles[2 * b].wait()
            handles[2 * b + 1].wait()

            @pl.when(nxt < NB_S)
            def _():
                pltpu.async_copy(e2_h.at[0, s, nxt], ring.at[b], isem[b])
        return carry

    lax.fori_loop(0, _A_ROUNDS, round_body, 0)
    plsc.subcore_barrier()

    pltpu.sync_copy(acc_o.at[pl.ds(r0, ROWS_PER_TILE)], nsbuf)
    pltpu.sync_copy(acc_i.at[pl.ds(r0, ROWS_PER_TILE)], ndbuf)

    def norm_body(r, carry):
        nsbuf[r, :] = _rsqrt16(nsbuf[r, :])
        ndbuf[r, :] = _rsqrt16(ndbuf[r, :])
        return carry

    lax.fori_loop(0, ROWS_PER_TILE, norm_body, 0)

    @pl.when(c == 0)
    def _():
        pltpu.sync_copy(nsbuf, ns_h.at[pl.ds(r0, ROWS_PER_TILE)])
        pltpu.sync_copy(ndbuf, nd_h.at[pl.ds(r0, ROWS_PER_TILE)])

    for k in range(ROWS_PER_TILE // 80):
        row0 = r0 + k * 80

        @pl.when(row0 < N_NODES)
        def _():
            pltpu.sync_copy(x_h.at[pl.ds(row0, 80)], xbuf)

            def scale_body(rr, carry):
                nsv = nsbuf[k * 80 + rr, :]

                @pl.when(c == 0)
                def _():
                    for q in range(4):
                        obuf[rr, pl.ds(16 * q, 16)] = (
                            xbuf[rr, pl.ds(16 * q, 16)] * nsv)

                @pl.when(c == 1)
                def _():
                    for q in range(4):
                        obuf[rr, pl.ds(16 * q, 16)] = (
                            xbuf[rr, pl.ds(64 + 16 * q, 16)] * nsv)
                return carry

            lax.fori_loop(0, 80, scale_body, 0)
            pltpu.sync_copy(obuf, xs_h.at[pl.ds(c * NPAD + row0, 80)])


_B_NBUF = 5
_B_ROUNDS = NB_S // _B_NBUF


@functools.partial(
    pl.kernel,
    out_type=jax.ShapeDtypeStruct((NC, NPAD, 64), jnp.float32),
    mesh=_mesh,
    scratch_types=[
        pltpu.VMEM((_B_NBUF, 2, BATCH_S), jnp.int32),
        pltpu.VMEM((_B_NBUF, BATCH_S, 64), jnp.float32),
        pltpu.VMEM_SHARED((NPAD, 64), jnp.float32),
    ] + [pltpu.SemaphoreType.DMA] * (3 * _B_NBUF),
    compiler_params=_sc_params,
)
def _sc_agg_colsplit(e2_h, tbl_h, z_h, out_h, ring, rows, acc, *sems):
    gsem = sems[:_B_NBUF]
    ssem = sems[_B_NBUF:2 * _B_NBUF]
    isem = sems[2 * _B_NBUF:]
    c = lax.axis_index("c")
    s = lax.axis_index("s")
    r0 = s * ROWS_PER_TILE
    for r in range(ROWS_PER_TILE // 128):
        pltpu.sync_copy(z_h, acc.at[pl.ds(r0 + r * 128, 128)])
    plsc.subcore_barrier()

    for b in range(_B_NBUF):
        pltpu.async_copy(e2_h.at[c, s, b], ring.at[b], isem[b])
    for b in range(_B_NBUF):
        pltpu.make_async_copy(e2_h.at[c, s, b], ring.at[b], isem[b]).wait()
        pltpu.async_copy(tbl_h.at[ring.at[b, 0]], rows.at[b], gsem[b])

    def round_body(g, carry):
        handles = []
        for b in range(_B_NBUF):
            i = g * _B_NBUF + b
            nxt = i + _B_NBUF
            pltpu.make_async_copy(
                tbl_h.at[ring.at[b, 0]], rows.at[b], gsem[b]).wait()
            handles.append(pltpu.async_copy(
                rows.at[b], acc.at[ring.at[b, 1]], ssem[b], add=True))

            @pl.when(nxt < NB_S)
            def _():
                pltpu.async_copy(e2_h.at[c, s, nxt], ring.at[b], isem[b])
        for b in range(_B_NBUF):
            i = g * _B_NBUF + b
            nxt = i + _B_NBUF
            handles[b].wait()

            @pl.when(nxt < NB_S)
            def _():
                pltpu.make_async_copy(
                    e2_h.at[c, s, nxt], ring.at[b], isem[b]).wait()
                pltpu.async_copy(
                    tbl_h.at[ring.at[b, 0]], rows.at[b], gsem[b])
        return carry

    lax.fori_loop(0, _B_ROUNDS, round_body, 0)
    plsc.subcore_barrier()
    pltpu.sync_copy(acc.at[pl.ds(r0, ROWS_PER_TILE)],
                    out_h.at[c, pl.ds(r0, ROWS_PER_TILE)])


_C_NBUF = 5
_C_ROUNDS = NB_W // _C_NBUF


@functools.partial(
    pl.kernel,
    out_type=jax.ShapeDtypeStruct((NC, NPAD, 32), jnp.float32),
    mesh=_mesh,
    scratch_types=[
        pltpu.VMEM((NB_W, BATCH), jnp.int32),
        pltpu.VMEM((NB_W, BATCH), jnp.int32),
        pltpu.VMEM((_C_NBUF, BATCH, 32), jnp.float32),
        pltpu.VMEM_SHARED((NPAD, 32), jnp.float32),
    ] + [pltpu.SemaphoreType.DMA] * (2 * _C_NBUF),
    compiler_params=_sc_params,
)
def _sc_agg32(src_h, dst_h, tbl_h, z_h, out_h,
              idx_g, idx_c, rows, acc, *sems):
    gsem = sems[:_C_NBUF]
    ssem = sems[_C_NBUF:]
    c = lax.axis_index("c")
    s = lax.axis_index("s")
    wid = c * NS + s
    r0 = s * ROWS_PER_TILE
    for r in range(ROWS_PER_TILE // 128):
        pltpu.sync_copy(z_h, acc.at[pl.ds(r0 + r * 128, 128)])
    pltpu.sync_copy(src_h.at[wid], idx_g)
    pltpu.sync_copy(dst_h.at[wid], idx_c)
    plsc.subcore_barrier()

    for b in range(_C_NBUF):
        pltpu.async_copy(tbl_h.at[idx_g.at[b]], rows.at[b], gsem[b])

    def round_body(g, carry):
        handles = []
        for b in range(_C_NBUF):
            i = g * _C_NBUF + b
            pltpu.make_async_copy(
                tbl_h.at[idx_g.at[i]], rows.at[b], gsem[b]).wait()
            handles.append(pltpu.async_copy(
                rows.at[b], acc.at[idx_c.at[i]], ssem[b], add=True))
        for b in range(_C_NBUF):
            handles[b].wait()
            nxt = (g + 1) * _C_NBUF + b

            @pl.when(nxt < NB_W)
            def _():
                pltpu.async_copy(
                    tbl_h.at[idx_g.at[nxt]], rows.at[b], gsem[b])
        return carry

    lax.fori_loop(0, _C_ROUNDS, round_body, 0)
    plsc.subcore_barrier()
    pltpu.sync_copy(acc.at[pl.ds(r0, ROWS_PER_TILE)],
                    out_h.at[c, pl.ds(r0, ROWS_PER_TILE)])


def _tc_mid_body(a_ref, nd_ref, ns_ref, w1_ref, b1_ref, w2_ref, t_ref):
    agg = jnp.concatenate([a_ref[0], a_ref[1]], axis=1) * nd_ref[:, :1]
    h = jnp.dot(agg, w1_ref[...], preferred_element_type=jnp.float32)
    h = jnp.maximum(h + b1_ref[...][None, :], 0.0)
    t_ref[...] = jnp.dot(h * ns_ref[:, :1], w2_ref[...],
                         preferred_element_type=jnp.float32)


def _tc_mid(agg0, nd, ns, W1, b1, W2):
    blk = 1024
    grid = NPAD // blk
    return pl.pallas_call(
        _tc_mid_body,
        grid=(grid,),
        in_specs=[
            pl.BlockSpec((NC, blk, 64), lambda i: (0, i, 0)),
            pl.BlockSpec((blk, 16), lambda i: (i, 0)),
            pl.BlockSpec((blk, 16), lambda i: (i, 0)),
            pl.BlockSpec((128, 256), lambda i: (0, 0)),
            pl.BlockSpec((256,), lambda i: (0,)),
            pl.BlockSpec((256, 32), lambda i: (0, 0)),
        ],
        out_specs=pl.BlockSpec((blk, 32), lambda i: (i, 0)),
        out_shape=jax.ShapeDtypeStruct((NPAD, 32), jnp.float32),
    )(agg0, nd, ns, W1, b1, W2)


def _tc_final_body(a_ref, nd_ref, b2_ref, o_ref):
    agg = (a_ref[0] + a_ref[1]) * nd_ref[:, :1]
    o_ref[...] = jax.nn.sigmoid(agg + b2_ref[...][None, :])


def _tc_final(agg1, nd, b2):
    blk = 1024
    grid = NPAD // blk
    return pl.pallas_call(
        _tc_final_body,
        grid=(grid,),
        in_specs=[
            pl.BlockSpec((NC, blk, 32), lambda i: (0, i, 0)),
            pl.BlockSpec((blk, 16), lambda i: (i, 0)),
            pl.BlockSpec((32,), lambda i: (0,)),
        ],
        out_specs=pl.BlockSpec((blk, 32), lambda i: (i, 0)),
        out_shape=jax.ShapeDtypeStruct((NPAD, 32), jnp.float32),
    )(agg1, nd, b2)


def kernel(x, edge_index, W1, b1, W2, b2):
    src = edge_index[0].astype(jnp.int32)
    dst = edge_index[1].astype(jnp.int32)
    src_w = src.reshape(NW, NB_W, BATCH)
    dst_w = dst.reshape(NW, NB_W, BATCH)
    src_s = src.reshape(NS, NB_S, BATCH_S)
    dst_s = dst.reshape(NS, NB_S, BATCH_S)
    e2 = jnp.stack([jnp.stack([src_s, dst_s], axis=2),
                    jnp.stack([src_s + NPAD, dst_s], axis=2)])

    ones16 = jnp.ones((BATCH_S, 16), jnp.float32)
    z16 = jnp.zeros((ROWS_PER_TILE, 16), jnp.float32)
    xs_st, ns, nd = _sc_prep(e2, x, ones16, z16)

    z64 = jnp.zeros((128, 64), jnp.float32)
    agg0 = _sc_agg_colsplit(e2, xs_st, z64)

    t = _tc_mid(agg0, nd, ns, W1, b1, W2)

    z32 = jnp.zeros((128, 32), jnp.float32)
    agg1 = _sc_agg32(src_w, dst_w, t, z32)

    out = _tc_final(agg1, nd, b2)
    return out[:N_NODES]


# speedup vs baseline: 1.1395x; 1.1287x over previous
import functools

import jax
import jax.numpy as jnp
from jax import lax
from jax.experimental import pallas as pl
from jax.experimental.pallas import tpu as pltpu
from jax.experimental.pallas import tpu_sc as plsc

N_NODES = 10000
N_EDGES = 320000
NPAD = 10240

NC = 2
NS = 16
NW = NC * NS
BATCH = 80
ROWS_PER_TILE = NPAD // NS

EPT_W = N_EDGES // NW
NB_W = EPT_W // BATCH
EPT_S = N_EDGES // NS
NB_S = EPT_S // BATCH

_mesh = plsc.VectorSubcoreMesh(core_axis_name="c", subcore_axis_name="s")
_sc_params = pltpu.CompilerParams(use_tc_tiling_on_sc=False)


_DEG_NBUF = 5
_DEG_ROUNDS = NB_W // _DEG_NBUF


@functools.partial(
    pl.kernel,
    out_type=(
        jax.ShapeDtypeStruct((NC, NPAD, 8), jnp.float32),
        jax.ShapeDtypeStruct((NC, NPAD, 8), jnp.float32),
    ),
    mesh=_mesh,
    scratch_types=[
        pltpu.VMEM((NB_W, BATCH), jnp.int32),
        pltpu.VMEM((NB_W, BATCH), jnp.int32),
        pltpu.VMEM((BATCH, 8), jnp.float32),
        pltpu.VMEM_SHARED((NPAD, 8), jnp.float32),
        pltpu.VMEM_SHARED((NPAD, 8), jnp.float32),
    ] + [pltpu.SemaphoreType.DMA] * (2 * _DEG_NBUF),
    compiler_params=_sc_params,
)
def _sc_degrees(src_h, dst_h, ones_h, z8_h, out_o, out_i,
                idx_s, idx_d, ones_v, acc_o, acc_i, *sems):
    osem = sems[:_DEG_NBUF]
    isem = sems[_DEG_NBUF:]
    c = lax.axis_index("c")
    s = lax.axis_index("s")
    wid = c * NS + s
    r0 = s * ROWS_PER_TILE
    pltpu.sync_copy(z8_h, acc_o.at[pl.ds(r0, ROWS_PER_TILE)])
    pltpu.sync_copy(z8_h, acc_i.at[pl.ds(r0, ROWS_PER_TILE)])
    pltpu.sync_copy(src_h.at[wid], idx_s)
    pltpu.sync_copy(dst_h.at[wid], idx_d)
    pltpu.sync_copy(ones_h, ones_v)
    plsc.subcore_barrier()

    def round_body(g, carry):
        handles = []
        for b in range(_DEG_NBUF):
            i = g * _DEG_NBUF + b
            handles.append(pltpu.async_copy(
                ones_v, acc_o.at[idx_s.at[i]], osem[b], add=True))
            handles.append(pltpu.async_copy(
                ones_v, acc_i.at[idx_d.at[i]], isem[b], add=True))
        for h in handles:
            h.wait()
        return carry

    lax.fori_loop(0, _DEG_ROUNDS, round_body, 0)
    plsc.subcore_barrier()
    pltpu.sync_copy(acc_o.at[pl.ds(r0, ROWS_PER_TILE)],
                    out_o.at[c, pl.ds(r0, ROWS_PER_TILE)])
    pltpu.sync_copy(acc_i.at[pl.ds(r0, ROWS_PER_TILE)],
                    out_i.at[c, pl.ds(r0, ROWS_PER_TILE)])


_B_NBUF = 5
_B_ROUNDS = NB_S // _B_NBUF


@functools.partial(
    pl.kernel,
    out_type=jax.ShapeDtypeStruct((NC, NPAD, 64), jnp.float32),
    mesh=_mesh,
    scratch_types=[
        pltpu.VMEM((_B_NBUF, 2, BATCH), jnp.int32),
        pltpu.VMEM((_B_NBUF, BATCH, 64), jnp.float32),
        pltpu.VMEM_SHARED((NPAD, 64), jnp.float32),
    ] + [pltpu.SemaphoreType.DMA] * (3 * _B_NBUF),
    compiler_params=_sc_params,
)
def _sc_agg_colsplit(e2_h, tbl_h, z_h, out_h, ring, rows, acc, *sems):
    gsem = sems[:_B_NBUF]
    ssem = sems[_B_NBUF:2 * _B_NBUF]
    isem = sems[2 * _B_NBUF:]
    c = lax.axis_index("c")
    s = lax.axis_index("s")
    r0 = s * ROWS_PER_TILE
    for r in range(ROWS_PER_TILE // 128):
        pltpu.sync_copy(z_h, acc.at[pl.ds(r0 + r * 128, 128)])
    plsc.subcore_barrier()

    for b in range(_B_NBUF):
        pltpu.async_copy(e2_h.at[c, s, b], ring.at[b], isem[b])
    for b in range(_B_NBUF):
        pltpu.make_async_copy(e2_h.at[c, s, b], ring.at[b], isem[b]).wait()
        pltpu.async_copy(tbl_h.at[ring.at[b, 0]], rows.at[b], gsem[b])

    def round_body(g, carry):
        handles = []
        for b in range(_B_NBUF):
            i = g * _B_NBUF + b
            nxt = i + _B_NBUF
            pltpu.make_async_copy(
                tbl_h.at[ring.at[b, 0]], rows.at[b], gsem[b]).wait()
            handles.append(pltpu.async_copy(
                rows.at[b], acc.at[ring.at[b, 1]], ssem[b], add=True))

            @pl.when(nxt < NB_S)
            def _():
                pltpu.async_copy(e2_h.at[c, s, nxt], ring.at[b], isem[b])
        for b in range(_B_NBUF):
            i = g * _B_NBUF + b
            nxt = i + _B_NBUF
            handles[b].wait()

            @pl.when(nxt < NB_S)
            def _():
                pltpu.make_async_copy(
                    e2_h.at[c, s, nxt], ring.at[b], isem[b]).wait()
                pltpu.async_copy(
                    tbl_h.at[ring.at[b, 0]], rows.at[b], gsem[b])
        return carry

    lax.fori_loop(0, _B_ROUNDS, round_body, 0)
    plsc.subcore_barrier()
    pltpu.sync_copy(acc.at[pl.ds(r0, ROWS_PER_TILE)],
                    out_h.at[c, pl.ds(r0, ROWS_PER_TILE)])


_C_NBUF = 5
_C_ROUNDS = NB_W // _C_NBUF


@functools.partial(
    pl.kernel,
    out_type=jax.ShapeDtypeStruct((NC, NPAD, 32), jnp.float32),
    mesh=_mesh,
    scratch_types=[
        pltpu.VMEM((NB_W, BATCH), jnp.int32),
        pltpu.VMEM((NB_W, BATCH), jnp.int32),
        pltpu.VMEM((_C_NBUF, BATCH, 32), jnp.float32),
        pltpu.VMEM_SHARED((NPAD, 32), jnp.float32),
    ] + [pltpu.SemaphoreType.DMA] * (2 * _C_NBUF),
    compiler_params=_sc_params,
)
def _sc_agg32(src_h, dst_h, tbl_h, z_h, out_h,
              idx_g, idx_c, rows, acc, *sems):
    gsem = sems[:_C_NBUF]
    ssem = sems[_C_NBUF:]
    c = lax.axis_index("c")
    s = lax.axis_index("s")
    wid = c * NS + s
    r0 = s * ROWS_PER_TILE
    for r in range(ROWS_PER_TILE // 128):
        pltpu.sync_copy(z_h, acc.at[pl.ds(r0 + r * 128, 128)])
    pltpu.sync_copy(src_h.at[wid], idx_g)
    pltpu.sync_copy(dst_h.at[wid], idx_c)
    plsc.subcore_barrier()

    for b in range(_C_NBUF):
        pltpu.async_copy(tbl_h.at[idx_g.at[b]], rows.at[b], gsem[b])

    def round_body(g, carry):
        handles = []
        for b in range(_C_NBUF):
            i = g * _C_NBUF + b
            pltpu.make_async_copy(
                tbl_h.at[idx_g.at[i]], rows.at[b], gsem[b]).wait()
            handles.append(pltpu.async_copy(
                rows.at[b], acc.at[idx_c.at[i]], ssem[b], add=True))
        for b in range(_C_NBUF):
            handles[b].wait()
            nxt = (g + 1) * _C_NBUF + b

            @pl.when(nxt < NB_W)
            def _():
                pltpu.async_copy(
                    tbl_h.at[idx_g.at[nxt]], rows.at[b], gsem[b])
        return carry

    lax.fori_loop(0, _C_ROUNDS, round_body, 0)
    plsc.subcore_barrier()
    pltpu.sync_copy(acc.at[pl.ds(r0, ROWS_PER_TILE)],
                    out_h.at[c, pl.ds(r0, ROWS_PER_TILE)])


def _tc_norms_body(dego_ref, degi_ref, x_ref, xs_ref, ns_ref, nd_ref):
    do = (dego_ref[0] + dego_ref[1])[:, :1]
    di = (degi_ref[0] + degi_ref[1])[:, :1]
    ns = lax.rsqrt(jnp.maximum(do, 1.0))
    nd = lax.rsqrt(jnp.maximum(di, 1.0))
    ns_ref[...] = ns
    nd_ref[...] = nd
    xs = x_ref[...] * ns
    xs_ref[0] = xs[:, :64]
    xs_ref[1] = xs[:, 64:]


def _tc_norms(deg_o, deg_i, x_pad):
    blk = 2048
    grid = NPAD // blk
    return pl.pallas_call(
        _tc_norms_body,
        grid=(grid,),
        in_specs=[
            pl.BlockSpec((NC, blk, 8), lambda i: (0, i, 0)),
            pl.BlockSpec((NC, blk, 8), lambda i: (0, i, 0)),
            pl.BlockSpec((blk, 128), lambda i: (i, 0)),
        ],
        out_specs=[
            pl.BlockSpec((2, blk, 64), lambda i: (0, i, 0)),
            pl.BlockSpec((blk, 1), lambda i: (i, 0)),
            pl.BlockSpec((blk, 1), lambda i: (i, 0)),
        ],
        out_shape=[
            jax.ShapeDtypeStruct((2, NPAD, 64), jnp.float32),
            jax.ShapeDtypeStruct((NPAD, 1), jnp.float32),
            jax.ShapeDtypeStruct((NPAD, 1), jnp.float32),
        ],
    )(deg_o, deg_i, x_pad)


def _tc_mid_body(a_ref, nd_ref, ns_ref, w1_ref, b1_ref, w2_ref, t_ref):
    agg = jnp.concatenate([a_ref[0], a_ref[1]], axis=1) * nd_ref[...]
    h = jnp.dot(agg, w1_ref[...], preferred_element_type=jnp.float32)
    h = jnp.maximum(h + b1_ref[...][None, :], 0.0)
    t_ref[...] = jnp.dot(h * ns_ref[...], w2_ref[...],
                         preferred_element_type=jnp.float32)


def _tc_mid(agg0, nd, ns, W1, b1, W2):
    blk = 1024
    grid = NPAD // blk
    return pl.pallas_call(
        _tc_mid_body,
        grid=(grid,),
        in_specs=[
            pl.BlockSpec((NC, blk, 64), lambda i: (0, i, 0)),
            pl.BlockSpec((blk, 1), lambda i: (i, 0)),
            pl.BlockSpec((blk, 1), lambda i: (i, 0)),
            pl.BlockSpec((128, 256), lambda i: (0, 0)),
            pl.BlockSpec((256,), lambda i: (0,)),
            pl.BlockSpec((256, 32), lambda i: (0, 0)),
        ],
        out_specs=pl.BlockSpec((blk, 32), lambda i: (i, 0)),
        out_shape=jax.ShapeDtypeStruct((NPAD, 32), jnp.float32),
    )(agg0, nd, ns, W1, b1, W2)


def _tc_final_body(a_ref, nd_ref, b2_ref, o_ref):
    agg = (a_ref[0] + a_ref[1]) * nd_ref[...]
    o_ref[...] = jax.nn.sigmoid(agg + b2_ref[...][None, :])


def _tc_final(agg1, nd, b2):
    blk = 1024
    grid = NPAD // blk
    return pl.pallas_call(
        _tc_final_body,
        grid=(grid,),
        in_specs=[
            pl.BlockSpec((NC, blk, 32), lambda i: (0, i, 0)),
            pl.BlockSpec((blk, 1), lambda i: (i, 0)),
            pl.BlockSpec((32,), lambda i: (0,)),
        ],
        out_specs=pl.BlockSpec((blk, 32), lambda i: (i, 0)),
        out_shape=jax.ShapeDtypeStruct((NPAD, 32), jnp.float32),
    )(agg1, nd, b2)


def kernel(x, edge_index, W1, b1, W2, b2):
    src = edge_index[0].astype(jnp.int32)
    dst = edge_index[1].astype(jnp.int32)
    src_w = src.reshape(NW, NB_W, BATCH)
    dst_w = dst.reshape(NW, NB_W, BATCH)
    src_s = src.reshape(NS, NB_S, BATCH)
    dst_s = dst.reshape(NS, NB_S, BATCH)
    e2 = jnp.stack([jnp.stack([src_s, dst_s], axis=2),
                    jnp.stack([src_s + NPAD, dst_s], axis=2)])

    ones8 = jnp.ones((BATCH, 8), jnp.float32)
    z8 = jnp.zeros((ROWS_PER_TILE, 8), jnp.float32)
    deg_o, deg_i = _sc_degrees(src_w, dst_w, ones8, z8)

    x_pad = jnp.pad(x, ((0, NPAD - N_NODES), (0, 0)))
    xs_pair, ns, nd = _tc_norms(deg_o, deg_i, x_pad)

    z64 = jnp.zeros((128, 64), jnp.float32)
    agg0 = _sc_agg_colsplit(e2, xs_pair.reshape(2 * NPAD, 64), z64)

    t = _tc_mid(agg0, nd, ns, W1, b1, W2)

    z32 = jnp.zeros((128, 32), jnp.float32)
    agg1 = _sc_agg32(src_w, dst_w, t, z32)

    out = _tc_final(agg1, nd, b2)
    return out[:N_NODES]
